# Initial kernel scaffold; baseline (speedup 1.0000x reference)
import jax
import jax.numpy as jnp
from jax.experimental import pallas as pl


def kernel(x, edge_index, W1_l, b1, W1_r, W2_l, b2, W2_r):
    raise NotImplementedError("write your pallas kernel here")


# speedup vs baseline: 10.4257x; 10.4257x over previous
import functools

import jax
import jax.numpy as jnp
from jax import lax
from jax.experimental import pallas as pl
from jax.experimental.pallas import tpu as pltpu
from jax.experimental.pallas import tpu_sc as plsc

_L = 16
_BLK = 128
_NW = 32


def _mm_body(x_ref, w_ref, o_ref):
    o_ref[...] = jnp.dot(x_ref[...], w_ref[...],
                         preferred_element_type=jnp.float32)


def _tc_matmul(x, w):
    n = x.shape[0]
    return pl.pallas_call(
        _mm_body,
        out_shape=jax.ShapeDtypeStruct((n, w.shape[1]), jnp.float32),
    )(x, w)


def _layer2_body(agg_ref, deg_ref, z1_ref, b1_ref, w2_ref, o_ref):
    deg = jnp.maximum(deg_ref[0] + deg_ref[1], 1.0)
    mean1 = (agg_ref[0] + agg_ref[1]) / deg
    h = jnp.maximum(mean1 + b1_ref[...] + z1_ref[...], 0.0)
    o_ref[...] = jnp.dot(h, w2_ref[...], preferred_element_type=jnp.float32)


def _layer3_body(agg_ref, deg_ref, z2_ref, b2_ref, o_ref):
    deg = jnp.maximum(deg_ref[0] + deg_ref[1], 1.0)
    o_ref[...] = (agg_ref[0] + agg_ref[1]) / deg + b2_ref[...] + z2_ref[...]


def _make_segsum(n_pad, nblk_tile, with_deg):
    mesh = plsc.VectorSubcoreMesh(core_axis_name="c", subcore_axis_name="s")
    rps = n_pad // 16

    out_type = [jax.ShapeDtypeStruct((2, n_pad, _L), jnp.float32)]
    scratch = [
        pltpu.VMEM((nblk_tile, _BLK), jnp.int32),
        pltpu.VMEM((nblk_tile, _BLK), jnp.int32),
        pltpu.VMEM((_BLK, _L), jnp.float32),
        pltpu.VMEM((rps, _L), jnp.float32),
        pltpu.VMEM_SHARED((n_pad, _L), jnp.float32),
        pltpu.SemaphoreType.DMA,
    ]
    if with_deg:
        out_type.append(jax.ShapeDtypeStruct((2, n_pad, _L), jnp.float32))
        scratch += [
            pltpu.VMEM((_BLK, _L), jnp.float32),
            pltpu.VMEM_SHARED((n_pad, _L), jnp.float32),
        ]

    def body(y_hbm, src_hbm, dst_hbm, out_hbm, *rest):
        if with_deg:
            (degout_hbm, src_v, dst_v, rows_v, z_v, acc_sh, sem,
             ones_v, dacc_sh) = rest
        else:
            src_v, dst_v, rows_v, z_v, acc_sh, sem = rest
            degout_hbm = ones_v = dacc_sh = None

        c = lax.axis_index("c")
        s = lax.axis_index("s")
        wid = s * 2 + c

        @pl.loop(0, rps)
        def _(i):
            z_v[pl.ds(i, 1), :] = jnp.zeros((1, _L), jnp.float32)

        my_rows = pl.ds(s * rps, rps)
        pltpu.sync_copy(z_v, acc_sh.at[my_rows])
        if with_deg:
            pltpu.sync_copy(z_v, dacc_sh.at[my_rows])

            @pl.loop(0, _BLK)
            def _(i):
                ones_v[pl.ds(i, 1), :] = jnp.ones((1, _L), jnp.float32)

        start = wid * nblk_tile
        pltpu.sync_copy(src_hbm.at[pl.ds(start, nblk_tile)], src_v)
        pltpu.sync_copy(dst_hbm.at[pl.ds(start, nblk_tile)], dst_v)

        plsc.subcore_barrier()

        @pl.loop(0, nblk_tile)
        def _(j):
            pltpu.async_copy(y_hbm.at[src_v.at[j]], rows_v, sem).wait()
            pltpu.sync_copy(rows_v, acc_sh.at[dst_v.at[j]], add=True)
            if with_deg:
                pltpu.sync_copy(ones_v, dacc_sh.at[dst_v.at[j]], add=True)

        plsc.subcore_barrier()

        pltpu.sync_copy(acc_sh.at[my_rows], out_hbm.at[c, my_rows])
        if with_deg:
            pltpu.sync_copy(dacc_sh.at[my_rows], degout_hbm.at[c, my_rows])

    return pl.kernel(
        body,
        out_type=tuple(out_type) if with_deg else out_type[0],
        mesh=mesh,
        scratch_types=scratch,
        compiler_params=pltpu.CompilerParams(use_tc_tiling_on_sc=False),
    )


@jax.jit
def kernel(x, edge_index, W1_l, b1, W1_r, W2_l, b2, W2_r):
    n, d = x.shape
    h_dim = W1_l.shape[1]
    e = edge_index.shape[1]
    assert h_dim == _L and W2_l.shape[1] == _L

    blk_per_tile = -(-e // (_BLK * _NW))
    blk_per_tile = -(-blk_per_tile // 8) * 8
    e_pad = blk_per_tile * _BLK * _NW
    n_pad = -(-(n + 1) // 128) * 128
    src = jnp.concatenate(
        [edge_index[0], jnp.zeros((e_pad - e,), jnp.int32)]).reshape(-1, _BLK)
    dst = jnp.concatenate(
        [edge_index[1], jnp.full((e_pad - e,), n, jnp.int32)]).reshape(-1, _BLK)

    segsum_deg = _make_segsum(n_pad, blk_per_tile, with_deg=True)
    segsum = _make_segsum(n_pad, blk_per_tile, with_deg=False)

    xw = _tc_matmul(x, jnp.concatenate([W1_l, W1_r], axis=1))
    agg1p, degp = segsum_deg(xw[:, :_L], src, dst)
    degp = degp[:, :n, :]

    hw = pl.pallas_call(
        _layer2_body,
        out_shape=jax.ShapeDtypeStruct((n, 2 * _L), jnp.float32),
    )(agg1p[:, :n, :], degp, xw[:, _L:], b1.reshape(1, _L),
      jnp.concatenate([W2_l, W2_r], axis=1))

    agg2p = segsum(hw[:, :_L], src, dst)

    out = pl.pallas_call(
        _layer3_body,
        out_shape=jax.ShapeDtypeStruct((n, _L), jnp.float32),
    )(agg2p[:, :n, :], degp, hw[:, _L:], b2.reshape(1, _L))
    return out


# speedup vs baseline: 13.5330x; 1.2980x over previous
import functools

import jax
import jax.numpy as jnp
from jax import lax
from jax.experimental import pallas as pl
from jax.experimental.pallas import tpu as pltpu
from jax.experimental.pallas import tpu_sc as plsc

_L = 16
_BLK = 128
_NW = 32


def _mm_body(x_ref, w_ref, o_ref):
    o_ref[...] = jnp.dot(x_ref[...], w_ref[...],
                         preferred_element_type=jnp.float32)


def _tc_matmul(x, w):
    n = x.shape[0]
    return pl.pallas_call(
        _mm_body,
        out_shape=jax.ShapeDtypeStruct((n, w.shape[1]), jnp.float32),
    )(x, w)


def _layer2_body(agg_ref, deg_ref, z1_ref, b1_ref, w2_ref, o_ref):
    deg = jnp.maximum(deg_ref[0] + deg_ref[1], 1.0)
    mean1 = (agg_ref[0] + agg_ref[1]) / deg
    h = jnp.maximum(mean1 + b1_ref[...] + z1_ref[...], 0.0)
    o_ref[...] = jnp.dot(h, w2_ref[...], preferred_element_type=jnp.float32)


def _layer3_body(agg_ref, deg_ref, z2_ref, b2_ref, o_ref):
    deg = jnp.maximum(deg_ref[0] + deg_ref[1], 1.0)
    o_ref[...] = (agg_ref[0] + agg_ref[1]) / deg + b2_ref[...] + z2_ref[...]


def _make_segsum(n_pad, nblk_tile, with_deg):
    mesh = plsc.VectorSubcoreMesh(core_axis_name="c", subcore_axis_name="s")
    rps = n_pad // 16
    nbuf = 4
    assert nblk_tile % nbuf == 0

    out_type = [jax.ShapeDtypeStruct((2, n_pad, _L), jnp.float32)]
    scratch = [
        pltpu.VMEM((nblk_tile, _BLK), jnp.int32),
        pltpu.VMEM((nblk_tile, _BLK), jnp.int32),
        pltpu.VMEM((nbuf, _BLK, _L), jnp.float32),
        pltpu.VMEM((rps, _L), jnp.float32),
        pltpu.VMEM_SHARED((n_pad, _L), jnp.float32),
        pltpu.SemaphoreType.DMA((nbuf,)),
        pltpu.SemaphoreType.DMA((nbuf,)),
    ]
    if with_deg:
        out_type.append(jax.ShapeDtypeStruct((2, n_pad, _L), jnp.float32))
        scratch += [
            pltpu.VMEM((_BLK, _L), jnp.float32),
            pltpu.VMEM_SHARED((n_pad, _L), jnp.float32),
            pltpu.SemaphoreType.DMA((nbuf,)),
        ]

    def body(y_hbm, src_hbm, dst_hbm, out_hbm, *rest):
        if with_deg:
            (degout_hbm, src_v, dst_v, rows_v, z_v, acc_sh, sem_g, sem_s,
             ones_v, dacc_sh, sem_o) = rest
        else:
            src_v, dst_v, rows_v, z_v, acc_sh, sem_g, sem_s = rest
            degout_hbm = ones_v = dacc_sh = sem_o = None

        c = lax.axis_index("c")
        s = lax.axis_index("s")
        wid = s * 2 + c

        start = wid * nblk_tile
        pltpu.sync_copy(src_hbm.at[pl.ds(start, nblk_tile)], src_v)
        pltpu.sync_copy(dst_hbm.at[pl.ds(start, nblk_tile)], dst_v)
        for b in range(nbuf):
            pltpu.async_copy(y_hbm.at[src_v.at[b]], rows_v.at[b], sem_g.at[b])

        @pl.loop(0, rps)
        def _(i):
            z_v[pl.ds(i, 1), :] = jnp.zeros((1, _L), jnp.float32)

        my_rows = pl.ds(s * rps, rps)
        pltpu.sync_copy(z_v, acc_sh.at[my_rows])
        if with_deg:
            pltpu.sync_copy(z_v, dacc_sh.at[my_rows])

            @pl.loop(0, _BLK)
            def _(i):
                ones_v[pl.ds(i, 1), :] = jnp.ones((1, _L), jnp.float32)

        plsc.subcore_barrier()

        @pl.loop(0, nblk_tile // nbuf)
        def _(g):
            j0 = g * nbuf
            for b in range(nbuf):
                j = j0 + b
                pltpu.make_async_copy(
                    y_hbm.at[src_v.at[j]], rows_v.at[b], sem_g.at[b]).wait()
                pltpu.async_copy(
                    rows_v.at[b], acc_sh.at[dst_v.at[j]], sem_s.at[b],
                    add=True)
                if with_deg:
                    @pl.when(g > 0)
                    def _():
                        pltpu.make_async_copy(
                            ones_v, dacc_sh.at[dst_v.at[j]], sem_o.at[b]
                        ).wait()
                    pltpu.async_copy(
                        ones_v, dacc_sh.at[dst_v.at[j]], sem_o.at[b], add=True)
            for b in range(nbuf):
                j = j0 + b
                pltpu.make_async_copy(
                    rows_v.at[b], acc_sh.at[dst_v.at[j]], sem_s.at[b]).wait()

                @pl.when(j0 + nbuf < nblk_tile)
                def _():
                    pltpu.async_copy(y_hbm.at[src_v.at[j + nbuf]],
                                     rows_v.at[b], sem_g.at[b])

        if with_deg:
            for b in range(nbuf):
                pltpu.make_async_copy(
                    ones_v, dacc_sh.at[dst_v.at[b]], sem_o.at[b]).wait()

        plsc.subcore_barrier()

        pltpu.sync_copy(acc_sh.at[my_rows], out_hbm.at[c, my_rows])
        if with_deg:
            pltpu.sync_copy(dacc_sh.at[my_rows], degout_hbm.at[c, my_rows])

    return pl.kernel(
        body,
        out_type=tuple(out_type) if with_deg else out_type[0],
        mesh=mesh,
        scratch_types=scratch,
        compiler_params=pltpu.CompilerParams(use_tc_tiling_on_sc=False),
    )


@jax.jit
def kernel(x, edge_index, W1_l, b1, W1_r, W2_l, b2, W2_r):
    n, d = x.shape
    h_dim = W1_l.shape[1]
    e = edge_index.shape[1]
    assert h_dim == _L and W2_l.shape[1] == _L

    blk_per_tile = -(-e // (_BLK * _NW))
    blk_per_tile = -(-blk_per_tile // 8) * 8
    e_pad = blk_per_tile * _BLK * _NW
    n_pad = -(-(n + 1) // 128) * 128
    src = jnp.concatenate(
        [edge_index[0], jnp.zeros((e_pad - e,), jnp.int32)]).reshape(-1, _BLK)
    dst = jnp.concatenate(
        [edge_index[1], jnp.full((e_pad - e,), n, jnp.int32)]).reshape(-1, _BLK)

    segsum_deg = _make_segsum(n_pad, blk_per_tile, with_deg=True)
    segsum = _make_segsum(n_pad, blk_per_tile, with_deg=False)

    xw = _tc_matmul(x, jnp.concatenate([W1_l, W1_r], axis=1))
    agg1p, degp = segsum_deg(xw[:, :_L], src, dst)
    degp = degp[:, :n, :]

    hw = pl.pallas_call(
        _layer2_body,
        out_shape=jax.ShapeDtypeStruct((n, 2 * _L), jnp.float32),
    )(agg1p[:, :n, :], degp, xw[:, _L:], b1.reshape(1, _L),
      jnp.concatenate([W2_l, W2_r], axis=1))

    agg2p = segsum(hw[:, :_L], src, dst)

    out = pl.pallas_call(
        _layer3_body,
        out_shape=jax.ShapeDtypeStruct((n, _L), jnp.float32),
    )(agg2p[:, :n, :], degp, hw[:, _L:], b2.reshape(1, _L))
    return out


# speedup vs baseline: 18.4291x; 1.3618x over previous
import functools

import jax
import jax.numpy as jnp
from jax import lax
from jax.experimental import pallas as pl
from jax.experimental.pallas import tpu as pltpu
from jax.experimental.pallas import tpu_sc as plsc

_L = 16
_BLK = 128
_NW = 32


def _mm_body(x_ref, w_ref, o_ref):
    o_ref[...] = jnp.dot(x_ref[...], w_ref[...],
                         preferred_element_type=jnp.float32)


def _tc_matmul(x, w):
    n = x.shape[0]
    return pl.pallas_call(
        _mm_body,
        out_shape=jax.ShapeDtypeStruct((n, w.shape[1]), jnp.float32),
    )(x, w)


def _layer2_body(agg_ref, deg_ref, z1_ref, b1_ref, w2_ref, o_ref):
    deg = jnp.maximum(deg_ref[0] + deg_ref[1], 1.0)
    mean1 = (agg_ref[0] + agg_ref[1]) / deg
    h = jnp.maximum(mean1 + b1_ref[...] + z1_ref[...], 0.0)
    o_ref[...] = jnp.dot(h, w2_ref[...], preferred_element_type=jnp.float32)


def _layer3_body(agg_ref, deg_ref, z2_ref, b2_ref, o_ref):
    deg = jnp.maximum(deg_ref[0] + deg_ref[1], 1.0)
    o_ref[...] = (agg_ref[0] + agg_ref[1]) / deg + b2_ref[...] + z2_ref[...]


def _make_segsum(n_pad, nblk_tile, with_deg):
    mesh = plsc.VectorSubcoreMesh(core_axis_name="c", subcore_axis_name="s")
    rps = n_pad // 16
    nbuf = 8
    assert nblk_tile % nbuf == 0

    out_type = [jax.ShapeDtypeStruct((2, n_pad, _L), jnp.float32)]
    scratch = [
        pltpu.VMEM((nblk_tile, _BLK), jnp.int32),
        pltpu.VMEM((nblk_tile, _BLK), jnp.int32),
        pltpu.VMEM((nbuf, _BLK, _L), jnp.float32),
        pltpu.VMEM((rps, _L), jnp.float32),
        pltpu.VMEM_SHARED((n_pad, _L), jnp.float32),
        pltpu.VMEM_SHARED((n_pad, _L), jnp.float32),
        pltpu.SemaphoreType.DMA((nbuf,)),
        pltpu.SemaphoreType.DMA((nbuf,)),
    ]
    if with_deg:
        out_type.append(jax.ShapeDtypeStruct((2, n_pad, _L), jnp.float32))
        scratch += [
            pltpu.VMEM((_BLK, _L), jnp.float32),
            pltpu.VMEM_SHARED((n_pad, _L), jnp.float32),
            pltpu.SemaphoreType.DMA((nbuf,)),
        ]

    def body(y_hbm, src_hbm, dst_hbm, out_hbm, *rest):
        if with_deg:
            (degout_hbm, src_v, dst_v, rows_v, z_v, acc_sh, y_sh, sem_g,
             sem_s, ones_v, dacc_sh, sem_o) = rest
        else:
            src_v, dst_v, rows_v, z_v, acc_sh, y_sh, sem_g, sem_s = rest
            degout_hbm = ones_v = dacc_sh = sem_o = None

        c = lax.axis_index("c")
        s = lax.axis_index("s")
        wid = s * 2 + c

        start = wid * nblk_tile
        pltpu.sync_copy(src_hbm.at[pl.ds(start, nblk_tile)], src_v)
        pltpu.sync_copy(dst_hbm.at[pl.ds(start, nblk_tile)], dst_v)
        my_rows = pl.ds(s * rps, rps)
        pltpu.sync_copy(y_hbm.at[my_rows], y_sh.at[my_rows])

        @pl.loop(0, rps)
        def _(i):
            z_v[pl.ds(i, 1), :] = jnp.zeros((1, _L), jnp.float32)

        pltpu.sync_copy(z_v, acc_sh.at[my_rows])
        if with_deg:
            pltpu.sync_copy(z_v, dacc_sh.at[my_rows])

            @pl.loop(0, _BLK)
            def _(i):
                ones_v[pl.ds(i, 1), :] = jnp.ones((1, _L), jnp.float32)

        plsc.subcore_barrier()
        for b in range(nbuf):
            pltpu.async_copy(y_sh.at[src_v.at[b]], rows_v.at[b], sem_g.at[b])

        @pl.loop(0, nblk_tile // nbuf)
        def _(g):
            j0 = g * nbuf
            for b in range(nbuf):
                j = j0 + b
                pltpu.make_async_copy(
                    y_sh.at[src_v.at[j]], rows_v.at[b], sem_g.at[b]).wait()
                pltpu.async_copy(
                    rows_v.at[b], acc_sh.at[dst_v.at[j]], sem_s.at[b],
                    add=True)
                if with_deg:
                    @pl.when(g > 0)
                    def _():
                        pltpu.make_async_copy(
                            ones_v, dacc_sh.at[dst_v.at[j]], sem_o.at[b]
                        ).wait()
                    pltpu.async_copy(
                        ones_v, dacc_sh.at[dst_v.at[j]], sem_o.at[b], add=True)
            for b in range(nbuf):
                j = j0 + b
                pltpu.make_async_copy(
                    rows_v.at[b], acc_sh.at[dst_v.at[j]], sem_s.at[b]).wait()

                @pl.when(j0 + nbuf < nblk_tile)
                def _():
                    pltpu.async_copy(y_sh.at[src_v.at[j + nbuf]],
                                     rows_v.at[b], sem_g.at[b])

        if with_deg:
            for b in range(nbuf):
                pltpu.make_async_copy(
                    ones_v, dacc_sh.at[dst_v.at[b]], sem_o.at[b]).wait()

        plsc.subcore_barrier()

        pltpu.sync_copy(acc_sh.at[my_rows], out_hbm.at[c, my_rows])
        if with_deg:
            pltpu.sync_copy(dacc_sh.at[my_rows], degout_hbm.at[c, my_rows])

    return pl.kernel(
        body,
        out_type=tuple(out_type) if with_deg else out_type[0],
        mesh=mesh,
        scratch_types=scratch,
        compiler_params=pltpu.CompilerParams(use_tc_tiling_on_sc=False),
    )


@jax.jit
def kernel(x, edge_index, W1_l, b1, W1_r, W2_l, b2, W2_r):
    n, d = x.shape
    h_dim = W1_l.shape[1]
    e = edge_index.shape[1]
    assert h_dim == _L and W2_l.shape[1] == _L

    blk_per_tile = -(-e // (_BLK * _NW))
    blk_per_tile = -(-blk_per_tile // 8) * 8
    e_pad = blk_per_tile * _BLK * _NW
    n_pad = -(-(n + 1) // 128) * 128
    src = jnp.concatenate(
        [edge_index[0], jnp.zeros((e_pad - e,), jnp.int32)]).reshape(-1, _BLK)
    dst = jnp.concatenate(
        [edge_index[1], jnp.full((e_pad - e,), n, jnp.int32)]).reshape(-1, _BLK)

    segsum_deg = _make_segsum(n_pad, blk_per_tile, with_deg=True)
    segsum = _make_segsum(n_pad, blk_per_tile, with_deg=False)

    def pad_y(y):
        return jnp.concatenate(
            [y, jnp.zeros((n_pad - n, _L), jnp.float32)])

    xw = _tc_matmul(x, jnp.concatenate([W1_l, W1_r], axis=1))
    agg1p, degp = segsum_deg(pad_y(xw[:, :_L]), src, dst)
    degp = degp[:, :n, :]

    hw = pl.pallas_call(
        _layer2_body,
        out_shape=jax.ShapeDtypeStruct((n, 2 * _L), jnp.float32),
    )(agg1p[:, :n, :], degp, xw[:, _L:], b1.reshape(1, _L),
      jnp.concatenate([W2_l, W2_r], axis=1))

    agg2p = segsum(pad_y(hw[:, :_L]), src, dst)

    out = pl.pallas_call(
        _layer3_body,
        out_shape=jax.ShapeDtypeStruct((n, _L), jnp.float32),
    )(agg2p[:, :n, :], degp, hw[:, _L:], b2.reshape(1, _L))
    return out
